# baseline (speedup 1.0000x reference)
import functools

import jax
import jax.numpy as jnp
from jax import lax
from jax.experimental import pallas as pl
from jax.experimental.pallas import tpu as pltpu
from jax.experimental.pallas import tpu_sc as plsc

NODE_DIM = 128
EDGE_DIM = 16

_NUM_CORES = 2
_NUM_SUBCORES = 16
_NUM_WORKERS = _NUM_CORES * _NUM_SUBCORES

_BLK = 5000

_CHUNK = 400


def _node_body(xf_ref, atoms_ref, at_ref, w_ref, b_ref, out_ref):
    blk = xf_ref.shape[0]
    av = atoms_ref[0]
    rows = lax.broadcasted_iota(jnp.int32, (NODE_DIM, blk), 0)
    oh_t = (rows == av).astype(jnp.float32)
    pb = lax.dot_general(
        at_ref[:], w_ref[:, :NODE_DIM], (((1,), (1,)), ((), ())),
        preferred_element_type=jnp.float32)
    pb = pb + b_ref[:]
    g = lax.dot_general(
        oh_t, pb, (((0,), (0,)), ((), ())),
        preferred_element_type=jnp.float32)
    xw = lax.dot_general(
        xf_ref[:], w_ref[:, NODE_DIM:], (((1,), (1,)), ((), ())),
        preferred_element_type=jnp.float32)
    out_ref[:] = g + xw


def _node_pallas(x_features, atoms3d, atom_pad, W, b2):
    n = x_features.shape[0]
    nb = n // _BLK
    return pl.pallas_call(
        _node_body,
        grid=(nb,),
        in_specs=[
            pl.BlockSpec((_BLK, NODE_DIM), lambda i: (i, 0)),
            pl.BlockSpec((1, 1, _BLK), lambda i: (i, 0, 0)),
            pl.BlockSpec((NODE_DIM, NODE_DIM), lambda i: (0, 0)),
            pl.BlockSpec((NODE_DIM, 2 * NODE_DIM), lambda i: (0, 0)),
            pl.BlockSpec((1, NODE_DIM), lambda i: (0, 0)),
        ],
        out_specs=pl.BlockSpec((_BLK, NODE_DIM), lambda i: (i, 0)),
        out_shape=jax.ShapeDtypeStruct((n, NODE_DIM), jnp.float32),
    )(x_features, atoms3d, atom_pad, W, b2)


def _make_edge_gather(n_edges, n_rows):
    rows_per_w = n_edges // _NUM_WORKERS
    n_chunks = rows_per_w // _CHUNK
    groups = _CHUNK // 16
    mesh = plsc.VectorSubcoreMesh(core_axis_name="c", subcore_axis_name="s")

    chunks_per_super = 5
    super_rows = _CHUNK * chunks_per_super
    n_super = rows_per_w // super_rows

    @functools.partial(
        pl.kernel,
        mesh=mesh,
        out_type=jax.ShapeDtypeStruct((n_edges, EDGE_DIM), jnp.float32),
        scratch_types=[
            pltpu.VMEM((super_rows,), jnp.int32),
            pltpu.VMEM((super_rows,), jnp.int32),
            pltpu.VMEM((_CHUNK, EDGE_DIM), jnp.float32),
            pltpu.VMEM((_CHUNK, EDGE_DIM), jnp.float32),
            pltpu.VMEM((n_rows, EDGE_DIM), jnp.float32),
            pltpu.SemaphoreType.DMA,
            pltpu.SemaphoreType.DMA,
            pltpu.SemaphoreType.DMA,
            pltpu.SemaphoreType.DMA,
        ],
    )
    def edge_k(table_hbm, idx_hbm, out_hbm, idx0, idx1, rows0, rows1,
               tb_v, sem_i0, sem_i1, sem_o0, sem_o1):
        wid = lax.axis_index("s") * _NUM_CORES + lax.axis_index("c")
        base_w = wid * rows_per_w
        idx_b = (idx0, idx1)
        rows_b = (rows0, rows1)
        sem_i = (sem_i0, sem_i1)
        sem_o = (sem_o0, sem_o1)

        pltpu.sync_copy(table_hbm, tb_v)

        def fire_idx(s, b):
            pltpu.async_copy(
                idx_hbm.at[pl.ds(base_w + s * super_rows, super_rows)],
                idx_b[b], sem_i[b])

        def wait_idx(b):
            pltpu.make_async_copy(
                idx_hbm.at[pl.ds(0, super_rows)], idx_b[b], sem_i[b]).wait()

        def fire_out(t, b):
            pltpu.async_copy(
                rows_b[b],
                out_hbm.at[pl.ds(base_w + t * _CHUNK, _CHUNK)], sem_o[b])

        def wait_out(b):
            pltpu.make_async_copy(
                rows_b[b], out_hbm.at[pl.ds(0, _CHUNK)], sem_o[b]).wait()

        def compute(ib, c, ob):
            iv_ref, rv_ref = idx_b[ib], rows_b[ob]

            @plsc.parallel_loop(0, groups, unroll=2)
            def _group(g):
                iv = iv_ref[pl.ds(c * _CHUNK + g * 16, 16)]
                for e in range(16):
                    s = iv[e]
                    rv_ref[g * 16 + e, :] = tb_v[s, :]

        def super_block(s, parity, first=False, last=False):
            ib = parity
            if not last:
                fire_idx(s + 1, 1 - ib)
            wait_idx(ib)
            for c in range(chunks_per_super):
                t = s * chunks_per_super + c
                ob = (parity * chunks_per_super + c) % 2
                if not (first and c < 2):
                    wait_out(ob)
                compute(ib, c, ob)
                fire_out(t, ob)

        fire_idx(0, 0)
        super_block(0, 0, first=True)

        def body(i, carry):
            s1 = 2 * i + 1
            super_block(s1, 1)
            super_block(s1 + 1, 0)
            return carry

        lax.fori_loop(0, (n_super - 3) // 2, body, 0)

        super_block(n_super - 2, 1)
        super_block(n_super - 1, 0, last=True)
        wait_out(0)
        wait_out(1)

    return edge_k


def kernel(x_features, atoms, edge_attr, pos, atom_table, edge_table, W, b):
    del pos
    n = x_features.shape[0]
    atoms3d = atoms.astype(jnp.int32).reshape(n // _BLK, 1, _BLK)
    atom_pad = jnp.pad(atom_table, ((0, NODE_DIM - atom_table.shape[0]), (0, 0)))
    b2 = b.reshape(1, NODE_DIM)

    edge_embed = _make_edge_gather(edge_attr.shape[0], edge_table.shape[0])(
        edge_table, edge_attr.astype(jnp.int32))
    x_out = _node_pallas(x_features, atoms3d, atom_pad, W, b2)
    return (x_out, edge_embed)


# speedup vs baseline: 1.0019x; 1.0019x over previous
import functools

import jax
import jax.numpy as jnp
from jax import lax
from jax.experimental import pallas as pl
from jax.experimental.pallas import tpu as pltpu
from jax.experimental.pallas import tpu_sc as plsc

NODE_DIM = 128
EDGE_DIM = 16

_NUM_CORES = 2
_NUM_SUBCORES = 16
_NUM_WORKERS = _NUM_CORES * _NUM_SUBCORES

_BLK = 4000

_CHUNK = 400


def _node_body(xf_ref, atoms_ref, at_ref, w_ref, b_ref, out_ref):
    blk = xf_ref.shape[0]
    av = atoms_ref[0]
    rows = lax.broadcasted_iota(jnp.int32, (NODE_DIM, blk), 0)
    oh_t = (rows == av).astype(jnp.float32)
    pb = lax.dot_general(
        at_ref[:], w_ref[:, :NODE_DIM], (((1,), (1,)), ((), ())),
        preferred_element_type=jnp.float32)
    pb = pb + b_ref[:]
    g = lax.dot_general(
        oh_t, pb, (((0,), (0,)), ((), ())),
        preferred_element_type=jnp.float32)
    xw = lax.dot_general(
        xf_ref[:], w_ref[:, NODE_DIM:], (((1,), (1,)), ((), ())),
        preferred_element_type=jnp.float32)
    out_ref[:] = g + xw


def _node_pallas(x_features, atoms3d, atom_pad, W, b2):
    n = x_features.shape[0]
    nb = n // _BLK
    return pl.pallas_call(
        _node_body,
        grid=(nb,),
        in_specs=[
            pl.BlockSpec((_BLK, NODE_DIM), lambda i: (i, 0)),
            pl.BlockSpec((1, 1, _BLK), lambda i: (i, 0, 0)),
            pl.BlockSpec((NODE_DIM, NODE_DIM), lambda i: (0, 0)),
            pl.BlockSpec((NODE_DIM, 2 * NODE_DIM), lambda i: (0, 0)),
            pl.BlockSpec((1, NODE_DIM), lambda i: (0, 0)),
        ],
        out_specs=pl.BlockSpec((_BLK, NODE_DIM), lambda i: (i, 0)),
        out_shape=jax.ShapeDtypeStruct((n, NODE_DIM), jnp.float32),
    )(x_features, atoms3d, atom_pad, W, b2)


def _make_edge_gather(n_edges, n_rows):
    rows_per_w = n_edges // _NUM_WORKERS
    n_chunks = rows_per_w // _CHUNK
    groups = _CHUNK // 16
    mesh = plsc.VectorSubcoreMesh(core_axis_name="c", subcore_axis_name="s")

    chunks_per_super = 5
    super_rows = _CHUNK * chunks_per_super
    n_super = rows_per_w // super_rows

    @functools.partial(
        pl.kernel,
        mesh=mesh,
        out_type=jax.ShapeDtypeStruct((n_edges, EDGE_DIM), jnp.float32),
        scratch_types=[
            pltpu.VMEM((super_rows,), jnp.int32),
            pltpu.VMEM((super_rows,), jnp.int32),
            pltpu.VMEM((_CHUNK, EDGE_DIM), jnp.float32),
            pltpu.VMEM((_CHUNK, EDGE_DIM), jnp.float32),
            pltpu.VMEM((n_rows, EDGE_DIM), jnp.float32),
            pltpu.SemaphoreType.DMA,
            pltpu.SemaphoreType.DMA,
            pltpu.SemaphoreType.DMA,
            pltpu.SemaphoreType.DMA,
        ],
    )
    def edge_k(table_hbm, idx_hbm, out_hbm, idx0, idx1, rows0, rows1,
               tb_v, sem_i0, sem_i1, sem_o0, sem_o1):
        wid = lax.axis_index("s") * _NUM_CORES + lax.axis_index("c")
        base_w = wid * rows_per_w
        idx_b = (idx0, idx1)
        rows_b = (rows0, rows1)
        sem_i = (sem_i0, sem_i1)
        sem_o = (sem_o0, sem_o1)

        pltpu.sync_copy(table_hbm, tb_v)

        def fire_idx(s, b):
            pltpu.async_copy(
                idx_hbm.at[pl.ds(base_w + s * super_rows, super_rows)],
                idx_b[b], sem_i[b])

        def wait_idx(b):
            pltpu.make_async_copy(
                idx_hbm.at[pl.ds(0, super_rows)], idx_b[b], sem_i[b]).wait()

        def fire_out(t, b):
            pltpu.async_copy(
                rows_b[b],
                out_hbm.at[pl.ds(base_w + t * _CHUNK, _CHUNK)], sem_o[b])

        def wait_out(b):
            pltpu.make_async_copy(
                rows_b[b], out_hbm.at[pl.ds(0, _CHUNK)], sem_o[b]).wait()

        def compute(ib, c, ob):
            iv_ref, rv_ref = idx_b[ib], rows_b[ob]

            @plsc.parallel_loop(0, groups, unroll=2)
            def _group(g):
                iv = iv_ref[pl.ds(c * _CHUNK + g * 16, 16)]
                for e in range(16):
                    s = iv[e]
                    rv_ref[g * 16 + e, :] = tb_v[s, :]

        def super_block(s, parity, first=False, last=False):
            ib = parity
            if not last:
                fire_idx(s + 1, 1 - ib)
            wait_idx(ib)
            for c in range(chunks_per_super):
                t = s * chunks_per_super + c
                ob = (parity * chunks_per_super + c) % 2
                if not (first and c < 2):
                    wait_out(ob)
                compute(ib, c, ob)
                fire_out(t, ob)

        fire_idx(0, 0)
        super_block(0, 0, first=True)

        def body(i, carry):
            s1 = 2 * i + 1
            super_block(s1, 1)
            super_block(s1 + 1, 0)
            return carry

        lax.fori_loop(0, (n_super - 3) // 2, body, 0)

        super_block(n_super - 2, 1)
        super_block(n_super - 1, 0, last=True)
        wait_out(0)
        wait_out(1)

    return edge_k


def kernel(x_features, atoms, edge_attr, pos, atom_table, edge_table, W, b):
    del pos
    n = x_features.shape[0]
    atoms3d = atoms.astype(jnp.int32).reshape(n // _BLK, 1, _BLK)
    atom_pad = jnp.pad(atom_table, ((0, NODE_DIM - atom_table.shape[0]), (0, 0)))
    b2 = b.reshape(1, NODE_DIM)

    edge_embed = _make_edge_gather(edge_attr.shape[0], edge_table.shape[0])(
        edge_table, edge_attr.astype(jnp.int32))
    x_out = _node_pallas(x_features, atoms3d, atom_pad, W, b2)
    return (x_out, edge_embed)
